# baseline (speedup 1.0000x reference)
import functools

import jax
import jax.numpy as jnp
from jax import lax
from jax.experimental import pallas as pl
from jax.experimental.pallas import tpu as pltpu
from jax.experimental.pallas import tpu_sc as plsc

_NUM_USER = 10000
_NUM_ITEM = 10000
_VOCAB = 50000
_NW = 320000
_DIM_E = 64
_NUM_NEG = 4
_B = 4096
_TEMP = 0.1
_LR_LAMBDA = 0.5
_REG_WEIGHT = 1e-3

_D = 128
_NC = 2
_NS = 16
_KW = 128
_NBLK = _NW // _KW
_BLK_PER_W = 80
_ITEM_PAD = 10240
_ROWS_PER_SUB = _ITEM_PAD // _NS
_ZROWS = 64

_G = 1 + _NUM_NEG
_NIDX = _B * _G
_IDX_PER_W = _NIDX // (_NC * _NS)
_GBLK = 128


def _sc_gather_segsum(t_feat_table, word_ids, word_item_ids):
    mesh = plsc.VectorSubcoreMesh(core_axis_name="c", subcore_axis_name="s")

    @functools.partial(
        pl.kernel,
        out_type=jax.ShapeDtypeStruct((_NC * _ITEM_PAD, _D), jnp.float32),
        mesh=mesh,
        scratch_types=[
            pltpu.VMEM((_ZROWS, _D), jnp.float32),
            pltpu.VMEM((_KW,), jnp.int32),
            pltpu.VMEM((_KW,), jnp.int32),
            pltpu.VMEM((_KW,), jnp.int32),
            pltpu.VMEM((_KW,), jnp.int32),
            pltpu.VMEM((_KW, _D), jnp.float32),
            pltpu.VMEM((_KW, _D), jnp.float32),
            pltpu.VMEM_SHARED((_ITEM_PAD, _D), jnp.float32),
            pltpu.SemaphoreType.DMA,
            pltpu.SemaphoreType.DMA,
            pltpu.SemaphoreType.DMA,
            pltpu.SemaphoreType.DMA,
            pltpu.SemaphoreType.DMA,
            pltpu.SemaphoreType.DMA,
        ],
    )
    def k(tab_hbm, wid_hbm, seg_hbm, out_hbm, zbuf, idx0, idx1, sg0, sg1,
          rows0, rows1, acc, isem0, isem1, ssem0, ssem1, gsem0, gsem1):
        c = lax.axis_index("c")
        s = lax.axis_index("s")
        w = c * _NS + s

        @pl.loop(0, _ZROWS)
        def _(r):
            for ch in range(_D // 16):
                zbuf[r, pl.ds(ch * 16, 16)] = jnp.zeros((16,), jnp.float32)

        row0 = s * _ROWS_PER_SUB
        for kk in range(_ROWS_PER_SUB // _ZROWS):
            pltpu.sync_copy(zbuf, acc.at[pl.ds(row0 + kk * _ZROWS, _ZROWS)])
        plsc.subcore_barrier()

        idxs = (idx0, idx1)
        sgs = (sg0, sg1)
        rows = (rows0, rows1)
        isems = (isem0, isem1)
        ssems = (ssem0, ssem1)
        gsems = (gsem0, gsem1)

        def valid(t):
            return w + 32 * t < _NBLK

        def fetch_start(t, b):
            @pl.when(valid(t))
            def _():
                base = (w + 32 * t) * _KW
                pltpu.async_copy(wid_hbm.at[pl.ds(base, _KW)], idxs[b],
                                 isems[b])
                pltpu.async_copy(seg_hbm.at[pl.ds(base, _KW)], sgs[b],
                                 ssems[b])

        def fetch_wait(t, b):
            @pl.when(valid(t))
            def _():
                base = (w + 32 * t) * _KW
                pltpu.make_async_copy(wid_hbm.at[pl.ds(base, _KW)], idxs[b],
                                      isems[b]).wait()
                pltpu.make_async_copy(seg_hbm.at[pl.ds(base, _KW)], sgs[b],
                                      ssems[b]).wait()

        def gather_start(t, b):
            @pl.when(valid(t))
            def _():
                pltpu.async_copy(tab_hbm.at[idxs[b]], rows[b], gsems[b])

        def gather_wait(t, b):
            @pl.when(valid(t))
            def _():
                pltpu.make_async_copy(tab_hbm.at[idxs[b]], rows[b],
                                      gsems[b]).wait()

        def scatter(t, b):
            @pl.when(valid(t))
            def _():
                pltpu.sync_copy(rows[b], acc.at[sgs[b]], add=True)

        fetch_start(0, 0)
        fetch_wait(0, 0)
        gather_start(0, 0)
        fetch_start(1, 1)

        @pl.loop(0, _BLK_PER_W // 2)
        def _(kk):
            for h in range(2):
                t = 2 * kk + h
                gather_wait(t, h)
                fetch_wait(t + 1, 1 - h)
                gather_start(t + 1, 1 - h)
                scatter(t, h)
                fetch_start(t + 2, h)

        plsc.subcore_barrier()

        pltpu.sync_copy(
            acc.at[pl.ds(row0, _ROWS_PER_SUB)],
            out_hbm.at[pl.ds(c * _ITEM_PAD + row0, _ROWS_PER_SUB)])

    return k(t_feat_table, word_ids, word_item_ids)


def _tc_mlp(partials, v_feat, W1, b1, W2, b2):
    def body(p_ref, v_ref, w1_ref, b1_ref, w2_ref, b2_ref, out_ref):
        summed = (p_ref[: _NUM_ITEM, :]
                  + p_ref[_ITEM_PAD:_ITEM_PAD + _NUM_ITEM, :])
        tn = jnp.sqrt(jnp.sum(summed * summed, axis=1, keepdims=True))
        t_agg = summed / jnp.maximum(tn, 1e-12)
        v = v_ref[...]
        vn = jnp.sqrt(jnp.sum(v * v, axis=1, keepdims=True))
        v = v / jnp.maximum(vn, 1e-12)
        feat = jnp.concatenate([v, t_agg], axis=1)
        h = lax.dot_general(
            feat, w1_ref[...], (((1,), (1,)), ((), ())),
            preferred_element_type=jnp.float32,
            precision=lax.Precision.HIGHEST) + b1_ref[...]
        h = jnp.where(h > 0, h, 0.01 * h)
        f = lax.dot_general(
            h, w2_ref[...], (((1,), (1,)), ((), ())),
            preferred_element_type=jnp.float32,
            precision=lax.Precision.HIGHEST) + b2_ref[...]
        out_ref[...] = jnp.concatenate(
            [f, jnp.zeros((_NUM_ITEM, _D - _DIM_E), jnp.float32)], axis=1)

    return pl.pallas_call(
        body,
        out_shape=jax.ShapeDtypeStruct((_NUM_ITEM, _D), jnp.float32),
    )(partials, v_feat, W1, b1.reshape(1, -1), W2, b2.reshape(1, -1))


def _sc_gather3(id_embedding, feature, user_tensor, item_idx, item_tensor):
    mesh = plsc.VectorSubcoreMesh(core_axis_name="c", subcore_axis_name="s")

    @functools.partial(
        pl.kernel,
        out_type=(jax.ShapeDtypeStruct((_B, _D), jnp.float32),
                  jax.ShapeDtypeStruct((_NIDX, _D), jnp.float32),
                  jax.ShapeDtypeStruct((_NIDX, _D), jnp.float32)),
        mesh=mesh,
        scratch_types=[
            pltpu.VMEM((_GBLK,), jnp.int32),
            pltpu.VMEM((_GBLK, _D), jnp.float32),
            pltpu.SemaphoreType.DMA,
        ],
    )
    def k(emb_hbm, feat_hbm, ui_hbm, ii_hbm, fi_hbm, ue_hbm, ie_hbm, fe_hbm,
          idx_v, rows_v, sem):
        c = lax.axis_index("c")
        s = lax.axis_index("s")
        w = c * _NS + s

        ubase = w * (_B // (_NC * _NS))
        pltpu.sync_copy(ui_hbm.at[pl.ds(ubase, _GBLK)], idx_v)
        pltpu.async_copy(emb_hbm.at[idx_v], rows_v, sem).wait()
        pltpu.sync_copy(rows_v, ue_hbm.at[pl.ds(ubase, _GBLK)])

        base0 = w * _IDX_PER_W
        for tab, idxs, out in ((emb_hbm, ii_hbm, ie_hbm),
                               (feat_hbm, fi_hbm, fe_hbm)):
            @pl.loop(0, _IDX_PER_W // _GBLK)
            def _(b):
                base = base0 + b * _GBLK
                pltpu.sync_copy(idxs.at[pl.ds(base, _GBLK)], idx_v)
                pltpu.async_copy(tab.at[idx_v], rows_v, sem).wait()
                pltpu.sync_copy(rows_v, out.at[pl.ds(base, _GBLK)])

    return k(id_embedding, feature, user_tensor, item_idx, item_tensor)


def _tc_loss(user_e, item_e5, feat_e5, id_embedding):
    def body(ue_ref, ie_ref, fe_ref, emb_ref, out_ref):
        ue = ue_ref[...]
        ie = ie_ref[...]
        fe = fe_ref[...]

        def contrast(anchor_cols, all_cols):
            es = []
            for j in range(_G):
                a = anchor_cols(j)
                v = all_cols(j)
                sj = jnp.sum(a * v, axis=1, keepdims=True) * (1.0 / _TEMP)
                es.append(jnp.exp(sj))
            tot = es[0] + es[1] + es[2] + es[3] + es[4]
            frac = es[0] / (tot + 1e-8)
            return jnp.mean(-jnp.log(frac + 1e-8))

        loss1 = contrast(lambda j: ue,
                         lambda j: ie[:, j * _D:(j + 1) * _D])
        loss2 = contrast(lambda j: ie[:, j * _D:(j + 1) * _D],
                         lambda j: fe[:, j * _D:(j + 1) * _D])
        emb = emb_ref[...]
        reg = _REG_WEIGHT * jnp.mean(emb * emb)
        out_ref[...] = jnp.broadcast_to(loss1 + _LR_LAMBDA * loss2 + reg,
                                        (1, 1))

    return pl.pallas_call(
        body,
        out_shape=jax.ShapeDtypeStruct((1, 1), jnp.float32),
    )(user_e, item_e5, feat_e5, id_embedding)


def kernel(v_feat, t_feat_table, W1, b1, W2, b2, id_embedding,
           word_item_ids, word_ids, user_tensor, item_tensor):
    wid_t = word_ids.reshape(_KW, _NBLK).transpose(1, 0).reshape(-1)
    seg_t = word_item_ids.reshape(_KW, _NBLK).transpose(1, 0).reshape(-1)
    partials = _sc_gather_segsum(t_feat_table, wid_t, seg_t)
    feature = _tc_mlp(partials, v_feat, W1, b1, W2, b2)
    item_idx = item_tensor + _NUM_USER
    emb_pad = jnp.pad(id_embedding, ((0, 0), (0, _D - _DIM_E)))
    user_e, item_e, feat_items = _sc_gather3(
        emb_pad, feature, user_tensor, item_idx, item_tensor)
    loss = _tc_loss(user_e,
                    item_e.reshape(_B, _G * _D),
                    feat_items.reshape(_B, _G * _D),
                    id_embedding)
    return loss.reshape(())


# speedup vs baseline: 1.0093x; 1.0093x over previous
import functools

import jax
import jax.numpy as jnp
from jax import lax
from jax.experimental import pallas as pl
from jax.experimental.pallas import tpu as pltpu
from jax.experimental.pallas import tpu_sc as plsc

_NUM_USER = 10000
_NUM_ITEM = 10000
_VOCAB = 50000
_NW = 320000
_DIM_E = 64
_NUM_NEG = 4
_B = 4096
_TEMP = 0.1
_LR_LAMBDA = 0.5
_REG_WEIGHT = 1e-3

_D = 128
_NC = 2
_NS = 16
_KW = 128
_NBLK = _NW // _KW
_BLK_PER_W = 84
_ITEM_PAD = 10240
_ROWS_PER_SUB = _ITEM_PAD // _NS
_ZROWS = 64

_G = 1 + _NUM_NEG
_NIDX = _B * _G
_IDX_PER_W = _NIDX // (_NC * _NS)
_GBLK = 128


def _sc_gather_segsum(t_feat_table, word_ids, word_item_ids):
    mesh = plsc.VectorSubcoreMesh(core_axis_name="c", subcore_axis_name="s")

    @functools.partial(
        pl.kernel,
        out_type=jax.ShapeDtypeStruct((_NC * _ITEM_PAD, _D), jnp.float32),
        mesh=mesh,
        scratch_types=[
            pltpu.VMEM((_ZROWS, _D), jnp.float32),
            pltpu.VMEM((_KW,), jnp.int32),
            pltpu.VMEM((_KW,), jnp.int32),
            pltpu.VMEM((_KW,), jnp.int32),
            pltpu.VMEM((_KW,), jnp.int32),
            pltpu.VMEM((_KW,), jnp.int32),
            pltpu.VMEM((_KW,), jnp.int32),
            pltpu.VMEM((_KW, _D), jnp.float32),
            pltpu.VMEM((_KW, _D), jnp.float32),
            pltpu.VMEM_SHARED((_ITEM_PAD, _D), jnp.float32),
            pltpu.SemaphoreType.DMA,
            pltpu.SemaphoreType.DMA,
            pltpu.SemaphoreType.DMA,
            pltpu.SemaphoreType.DMA,
            pltpu.SemaphoreType.DMA,
            pltpu.SemaphoreType.DMA,
            pltpu.SemaphoreType.DMA,
            pltpu.SemaphoreType.DMA,
            pltpu.SemaphoreType.DMA,
            pltpu.SemaphoreType.DMA,
        ],
    )
    def k(tab_hbm, wid_hbm, seg_hbm, out_hbm, zbuf, idx0, idx1, idx2,
          sg0, sg1, sg2, rows0, rows1, acc, isem0, isem1, isem2,
          ssem0, ssem1, ssem2, gsem0, gsem1, csem0, csem1):
        c = lax.axis_index("c")
        s = lax.axis_index("s")
        w = c * _NS + s

        @pl.loop(0, _ZROWS)
        def _(r):
            for ch in range(_D // 16):
                zbuf[r, pl.ds(ch * 16, 16)] = jnp.zeros((16,), jnp.float32)

        row0 = s * _ROWS_PER_SUB
        for kk in range(_ROWS_PER_SUB // _ZROWS):
            pltpu.sync_copy(zbuf, acc.at[pl.ds(row0 + kk * _ZROWS, _ZROWS)])
        plsc.subcore_barrier()

        idxs = (idx0, idx1, idx2)
        sgs = (sg0, sg1, sg2)
        rows = (rows0, rows1)
        isems = (isem0, isem1, isem2)
        ssems = (ssem0, ssem1, ssem2)
        gsems = (gsem0, gsem1)
        csems = (csem0, csem1)

        def valid(t):
            return w + 32 * t < _NBLK

        def fetch_start(t, i):
            @pl.when(valid(t))
            def _():
                base = (w + 32 * t) * _KW
                pltpu.async_copy(wid_hbm.at[pl.ds(base, _KW)], idxs[i],
                                 isems[i])
                pltpu.async_copy(seg_hbm.at[pl.ds(base, _KW)], sgs[i],
                                 ssems[i])

        def fetch_wait(t, i):
            @pl.when(valid(t))
            def _():
                base = (w + 32 * t) * _KW
                pltpu.make_async_copy(wid_hbm.at[pl.ds(base, _KW)], idxs[i],
                                      isems[i]).wait()
                pltpu.make_async_copy(seg_hbm.at[pl.ds(base, _KW)], sgs[i],
                                      ssems[i]).wait()

        def gather_start(t, r, i):
            @pl.when(valid(t))
            def _():
                pltpu.async_copy(tab_hbm.at[idxs[i]], rows[r], gsems[r])

        def gather_wait(t, r, i):
            @pl.when(valid(t))
            def _():
                pltpu.make_async_copy(tab_hbm.at[idxs[i]], rows[r],
                                      gsems[r]).wait()

        def scatter_start(t, r, i):
            @pl.when(valid(t))
            def _():
                pltpu.async_copy(rows[r], acc.at[sgs[i]], csems[r],
                                 add=True)

        def scatter_wait(t, r, i):
            @pl.when((t >= 0) & valid(t))
            def _():
                pltpu.make_async_copy(rows[r], acc.at[sgs[i]],
                                      csems[r]).wait()

        fetch_start(0, 0)
        fetch_wait(0, 0)
        gather_start(0, 0, 0)
        fetch_start(1, 1)

        @pl.loop(0, _BLK_PER_W // 6)
        def _(kk):
            for h in range(6):
                t = 6 * kk + h
                r = h % 2
                i = h % 3
                gather_wait(t, r, i)
                scatter_start(t, r, i)
                scatter_wait(t - 1, 1 - r, (h + 2) % 3)
                fetch_wait(t + 1, (h + 1) % 3)
                gather_start(t + 1, 1 - r, (h + 1) % 3)
                fetch_start(t + 2, (h + 2) % 3)

        plsc.subcore_barrier()

        pltpu.sync_copy(
            acc.at[pl.ds(row0, _ROWS_PER_SUB)],
            out_hbm.at[pl.ds(c * _ITEM_PAD + row0, _ROWS_PER_SUB)])

    return k(t_feat_table, word_ids, word_item_ids)


def _tc_mlp(partials, v_feat, W1, b1, W2, b2):
    def body(p_ref, v_ref, w1_ref, b1_ref, w2_ref, b2_ref, out_ref):
        summed = (p_ref[: _NUM_ITEM, :]
                  + p_ref[_ITEM_PAD:_ITEM_PAD + _NUM_ITEM, :])
        tn = jnp.sqrt(jnp.sum(summed * summed, axis=1, keepdims=True))
        t_agg = summed / jnp.maximum(tn, 1e-12)
        v = v_ref[...]
        vn = jnp.sqrt(jnp.sum(v * v, axis=1, keepdims=True))
        v = v / jnp.maximum(vn, 1e-12)
        feat = jnp.concatenate([v, t_agg], axis=1)
        h = lax.dot_general(
            feat, w1_ref[...], (((1,), (1,)), ((), ())),
            preferred_element_type=jnp.float32,
            precision=lax.Precision.HIGHEST) + b1_ref[...]
        h = jnp.where(h > 0, h, 0.01 * h)
        f = lax.dot_general(
            h, w2_ref[...], (((1,), (1,)), ((), ())),
            preferred_element_type=jnp.float32,
            precision=lax.Precision.HIGHEST) + b2_ref[...]
        out_ref[...] = jnp.concatenate(
            [f, jnp.zeros((_NUM_ITEM, _D - _DIM_E), jnp.float32)], axis=1)

    return pl.pallas_call(
        body,
        out_shape=jax.ShapeDtypeStruct((_NUM_ITEM, _D), jnp.float32),
    )(partials, v_feat, W1, b1.reshape(1, -1), W2, b2.reshape(1, -1))


def _sc_gather3(id_embedding, feature, user_tensor, item_idx, item_tensor):
    mesh = plsc.VectorSubcoreMesh(core_axis_name="c", subcore_axis_name="s")

    @functools.partial(
        pl.kernel,
        out_type=(jax.ShapeDtypeStruct((_B, _D), jnp.float32),
                  jax.ShapeDtypeStruct((_NIDX, _D), jnp.float32),
                  jax.ShapeDtypeStruct((_NIDX, _D), jnp.float32)),
        mesh=mesh,
        scratch_types=[
            pltpu.VMEM((_GBLK,), jnp.int32),
            pltpu.VMEM((_GBLK, _D), jnp.float32),
            pltpu.SemaphoreType.DMA,
        ],
    )
    def k(emb_hbm, feat_hbm, ui_hbm, ii_hbm, fi_hbm, ue_hbm, ie_hbm, fe_hbm,
          idx_v, rows_v, sem):
        c = lax.axis_index("c")
        s = lax.axis_index("s")
        w = c * _NS + s

        ubase = w * (_B // (_NC * _NS))
        pltpu.sync_copy(ui_hbm.at[pl.ds(ubase, _GBLK)], idx_v)
        pltpu.async_copy(emb_hbm.at[idx_v], rows_v, sem).wait()
        pltpu.sync_copy(rows_v, ue_hbm.at[pl.ds(ubase, _GBLK)])

        base0 = w * _IDX_PER_W
        for tab, idxs, out in ((emb_hbm, ii_hbm, ie_hbm),
                               (feat_hbm, fi_hbm, fe_hbm)):
            @pl.loop(0, _IDX_PER_W // _GBLK)
            def _(b):
                base = base0 + b * _GBLK
                pltpu.sync_copy(idxs.at[pl.ds(base, _GBLK)], idx_v)
                pltpu.async_copy(tab.at[idx_v], rows_v, sem).wait()
                pltpu.sync_copy(rows_v, out.at[pl.ds(base, _GBLK)])

    return k(id_embedding, feature, user_tensor, item_idx, item_tensor)


def _tc_loss(user_e, item_e5, feat_e5, id_embedding):
    def body(ue_ref, ie_ref, fe_ref, emb_ref, out_ref):
        ue = ue_ref[...]
        ie = ie_ref[...]
        fe = fe_ref[...]

        def contrast(anchor_cols, all_cols):
            es = []
            for j in range(_G):
                a = anchor_cols(j)
                v = all_cols(j)
                sj = jnp.sum(a * v, axis=1, keepdims=True) * (1.0 / _TEMP)
                es.append(jnp.exp(sj))
            tot = es[0] + es[1] + es[2] + es[3] + es[4]
            frac = es[0] / (tot + 1e-8)
            return jnp.mean(-jnp.log(frac + 1e-8))

        loss1 = contrast(lambda j: ue,
                         lambda j: ie[:, j * _D:(j + 1) * _D])
        loss2 = contrast(lambda j: ie[:, j * _D:(j + 1) * _D],
                         lambda j: fe[:, j * _D:(j + 1) * _D])
        emb = emb_ref[...]
        reg = _REG_WEIGHT * jnp.mean(emb * emb)
        out_ref[...] = jnp.broadcast_to(loss1 + _LR_LAMBDA * loss2 + reg,
                                        (1, 1))

    return pl.pallas_call(
        body,
        out_shape=jax.ShapeDtypeStruct((1, 1), jnp.float32),
    )(user_e, item_e5, feat_e5, id_embedding)


def kernel(v_feat, t_feat_table, W1, b1, W2, b2, id_embedding,
           word_item_ids, word_ids, user_tensor, item_tensor):
    partials = _sc_gather_segsum(t_feat_table, word_ids, word_item_ids)
    feature = _tc_mlp(partials, v_feat, W1, b1, W2, b2)
    item_idx = item_tensor + _NUM_USER
    emb_pad = jnp.pad(id_embedding, ((0, 0), (0, _D - _DIM_E)))
    user_e, item_e, feat_items = _sc_gather3(
        emb_pad, feature, user_tensor, item_idx, item_tensor)
    loss = _tc_loss(user_e,
                    item_e.reshape(_B, _G * _D),
                    feat_items.reshape(_B, _G * _D),
                    id_embedding)
    return loss.reshape(())
